# baseline (speedup 1.0000x reference)
import jax
import jax.numpy as jnp
from jax import lax
from jax.experimental import pallas as pl
from jax.experimental.pallas import tpu as pltpu
from jax.experimental.pallas import tpu_sc as plsc

C0, C1 = 20000, 60000
D0, D1, D2 = 1024, 256, 64
DCAT = D0 + D1 + 2 * D2
OUT_DIM = 1024

NC, NS = 2, 16
NW = NC * NS

CH0, CH1, CH2 = 64, 128, 128


def _gather_sc(tok, emb0, emb1, emb2p):
    T = tok.shape[0]
    tpw = T // NW
    n0 = tpw // CH0
    n1 = tpw // CH1
    n2 = tpw // CH2
    mesh = plsc.VectorSubcoreMesh(core_axis_name="c", subcore_axis_name="s")

    def body(tok_hbm, e0_hbm, e1_hbm, e2_hbm, out_hbm,
             tok_v, i0_v, i1_v, i2_v, b0_v, b1_v, b2_v, sem):
        wid = lax.axis_index("s") * NC + lax.axis_index("c")
        base = wid * tpw
        pltpu.sync_copy(tok_hbm.at[pl.ds(base, tpw)], tok_v)

        for j in range(tpw // 16):
            t = tok_v[pl.ds(j * 16, 16)]
            z = jnp.zeros((16,), jnp.int32)
            inb0 = t < C0
            inb1 = jnp.logical_and(t >= C0, t < C1)
            inb2 = t >= C1
            g0 = jnp.where(inb0, t, z)
            g1 = jnp.where(inb1, t - C0, z)
            g2 = jnp.where(inb2, lax.shift_right_logical(t - C1, 1), z)
            off = j * 16
            i0_v[off // CH0, pl.ds(off % CH0, 16)] = g0
            i1_v[off // CH1, pl.ds(off % CH1, 16)] = g1
            i2_v[off // CH2, pl.ds(off % CH2, 16)] = g2

        for c in range(n0):
            pltpu.async_copy(e0_hbm.at[i0_v.at[c]], b0_v, sem).wait()
            pltpu.sync_copy(
                b0_v, out_hbm.at[pl.ds(base + c * CH0, CH0), pl.ds(0, D0)])
        for c in range(n1):
            pltpu.async_copy(e1_hbm.at[i1_v.at[c]], b1_v, sem).wait()
            pltpu.sync_copy(
                b1_v, out_hbm.at[pl.ds(base + c * CH1, CH1), pl.ds(D0, D1)])
        for c in range(n2):
            pltpu.async_copy(e2_hbm.at[i2_v.at[c]], b2_v, sem).wait()
            pltpu.sync_copy(
                b2_v,
                out_hbm.at[pl.ds(base + c * CH2, CH2), pl.ds(D0 + D1, 2 * D2)])

    run = pl.kernel(
        body,
        out_type=jax.ShapeDtypeStruct((T, DCAT), jnp.float32),
        mesh=mesh,
        scratch_types=[
            pltpu.VMEM((tpw,), jnp.int32),
            pltpu.VMEM((n0, CH0), jnp.int32),
            pltpu.VMEM((n1, CH1), jnp.int32),
            pltpu.VMEM((n2, CH2), jnp.int32),
            pltpu.VMEM((CH0, D0), jnp.float32),
            pltpu.VMEM((CH1, D1), jnp.float32),
            pltpu.VMEM((CH2, 2 * D2), jnp.float32),
            pltpu.SemaphoreType.DMA,
        ],
    )
    return run(tok, emb0, emb1, emb2p)


def _matmul_tc(e, wcat, tok):
    T = e.shape[0]
    bm = 1024
    tok3 = tok.reshape(T // bm, bm, 1)

    def body(e_ref, w_ref, t_ref, o_ref):
        t = t_ref[0, :, :]
        sel_hi = jnp.logical_and(t >= C1, (t & 1) == 1)
        col = lax.broadcasted_iota(jnp.int32, (bm, 2 * D2), 1)
        keep = sel_hi == (col >= D2)
        e = e_ref[...]
        e01 = e[:, : D0 + D1].astype(jnp.bfloat16)
        epair = jnp.where(keep, e[:, D0 + D1:], 0.0).astype(jnp.bfloat16)
        w = w_ref[...].astype(jnp.bfloat16)
        acc = lax.dot_general(
            e01, w[:, : D0 + D1], (((1,), (1,)), ((), ())),
            preferred_element_type=jnp.float32)
        acc += lax.dot_general(
            epair, w[:, D0 + D1:], (((1,), (1,)), ((), ())),
            preferred_element_type=jnp.float32)
        o_ref[...] = acc

    return pl.pallas_call(
        body,
        grid=(T // bm,),
        in_specs=[
            pl.BlockSpec((bm, DCAT), lambda i: (i, 0)),
            pl.BlockSpec((OUT_DIM, DCAT), lambda i: (0, 0)),
            pl.BlockSpec((1, bm, 1), lambda i: (i, 0, 0)),
        ],
        out_specs=pl.BlockSpec((bm, OUT_DIM), lambda i: (i, 0)),
        out_shape=jax.ShapeDtypeStruct((T, OUT_DIM), jnp.float32),
    )(e, wcat, tok3)


def kernel(input, emb0, emb1, emb2, W0, W1, W2):
    B, L = input.shape
    tok = input.reshape(B * L)
    emb2p = emb2.reshape(emb2.shape[0] // 2, 2 * D2)
    wcat = jnp.concatenate([W0, W1, W2, W2], axis=1)
    e = _gather_sc(tok, emb0, emb1, emb2p)
    out = _matmul_tc(e, wcat, tok)
    return out.reshape(B, L, OUT_DIM)
